# Initial kernel scaffold; baseline (speedup 1.0000x reference)
import jax
import jax.numpy as jnp
from jax.experimental import pallas as pl


def kernel(indices, table):
    raise NotImplementedError("write your pallas kernel here")


# speedup vs baseline: 1.8328x; 1.8328x over previous
import functools

import jax
import jax.numpy as jnp
from jax import lax
from jax.experimental import pallas as pl
from jax.experimental.pallas import tpu as pltpu
from jax.experimental.pallas import tpu_sc as plsc

VOCAB = 1000000
DIM = 64
BATCH = 16384
HIST = 50
B = BATCH * HIST

NUM_CORES = 2
NUM_SUBCORES = 16
NW = NUM_CORES * NUM_SUBCORES
BPW = B // NW

CHUNK = 512
SUB = 128
NSUB = CHUNK // SUB
NCHUNK = BPW // CHUNK

_mesh = plsc.VectorSubcoreMesh(core_axis_name="c", subcore_axis_name="s")


@functools.partial(
    pl.kernel,
    mesh=_mesh,
    compiler_params=pltpu.CompilerParams(use_tc_tiling_on_sc=False),
    out_type=jax.ShapeDtypeStruct((B, DIM), jnp.float32),
    scratch_types=[
        pltpu.VMEM((BPW,), jnp.int32),
        pltpu.VMEM((CHUNK, DIM), jnp.float32),
        pltpu.SemaphoreType.DMA,
    ],
)
def _gather(idx_hbm, table_hbm, out_hbm, idx_v, rows_v, sem):
    wid = lax.axis_index("s") * NUM_CORES + lax.axis_index("c")
    base = wid * BPW
    pltpu.sync_copy(idx_hbm.at[pl.ds(base, BPW)], idx_v)

    def body(c, carry):
        off = c * CHUNK
        handles = []
        for j in range(NSUB):
            handles.append(
                pltpu.async_copy(
                    table_hbm.at[idx_v.at[pl.ds(off + j * SUB, SUB)]],
                    rows_v.at[pl.ds(j * SUB, SUB)],
                    sem,
                )
            )
        for h in handles:
            h.wait()
        pltpu.sync_copy(rows_v, out_hbm.at[pl.ds(base + off, CHUNK)])
        return carry

    lax.fori_loop(0, NCHUNK, body, 0)


def kernel(indices, table):
    flat = indices.reshape(B)
    out = _gather(flat, table)
    return out.reshape(BATCH, HIST, DIM)


# speedup vs baseline: 1.8724x; 1.0216x over previous
import functools

import jax
import jax.numpy as jnp
from jax import lax
from jax.experimental import pallas as pl
from jax.experimental.pallas import tpu as pltpu
from jax.experimental.pallas import tpu_sc as plsc

VOCAB = 1000000
DIM = 64
BATCH = 16384
HIST = 50
B = BATCH * HIST

NUM_CORES = 2
NUM_SUBCORES = 16
NW = NUM_CORES * NUM_SUBCORES
BPW = B // NW

CHUNK = 512
SUB = 128
NSUB = CHUNK // SUB
NCHUNK = BPW // CHUNK

_mesh = plsc.VectorSubcoreMesh(core_axis_name="c", subcore_axis_name="s")


@functools.partial(
    pl.kernel,
    mesh=_mesh,
    compiler_params=pltpu.CompilerParams(use_tc_tiling_on_sc=False),
    out_type=jax.ShapeDtypeStruct((B, DIM), jnp.float32),
    scratch_types=[
        pltpu.VMEM((BPW,), jnp.int32),
        pltpu.VMEM((CHUNK, DIM), jnp.float32),
        pltpu.VMEM((CHUNK, DIM), jnp.float32),
        pltpu.SemaphoreType.DMA,
        pltpu.SemaphoreType.DMA,
        pltpu.SemaphoreType.DMA,
        pltpu.SemaphoreType.DMA,
    ],
)
def _gather(idx_hbm, table_hbm, out_hbm, idx_v, rows0, rows1, g0, g1, s0, s1):
    wid = lax.axis_index("s") * NUM_CORES + lax.axis_index("c")
    base = wid * BPW
    pltpu.sync_copy(idx_hbm.at[pl.ds(base, BPW)], idx_v)

    rows = (rows0, rows1)
    gsem = (g0, g1)
    ssem = (s0, s1)

    def issue_gathers(c, b):
        off = c * CHUNK
        for j in range(NSUB):
            pltpu.async_copy(
                table_hbm.at[idx_v.at[pl.ds(off + j * SUB, SUB)]],
                rows[b].at[pl.ds(j * SUB, SUB)],
                gsem[b],
            )

    def wait_gathers(b):
        for j in range(NSUB):
            pltpu.make_async_copy(
                table_hbm.at[idx_v.at[pl.ds(j * SUB, SUB)]],
                rows[b].at[pl.ds(j * SUB, SUB)],
                gsem[b],
            ).wait()

    def wait_store(b):
        pltpu.make_async_copy(
            rows[b], out_hbm.at[pl.ds(base, CHUNK)], ssem[b]
        ).wait()

    issue_gathers(0, 0)

    def outer(i, carry):
        for b in range(2):
            c = 2 * i + b
            nb = 1 - b
            wait_gathers(b)
            pltpu.async_copy(
                rows[b], out_hbm.at[pl.ds(base + c * CHUNK, CHUNK)], ssem[b]
            )

            @pl.when(c > 0)
            def _():
                wait_store(nb)

            @pl.when(c + 1 < NCHUNK)
            def _():
                issue_gathers(c + 1, nb)

        return carry

    lax.fori_loop(0, NCHUNK // 2, outer, 0)
    wait_store(1)


def kernel(indices, table):
    flat = indices.reshape(B)
    out = _gather(flat, table)
    return out.reshape(BATCH, HIST, DIM)
